# Initial kernel scaffold; baseline (speedup 1.0000x reference)
import jax
import jax.numpy as jnp
from jax.experimental import pallas as pl


def kernel(x, encoding):
    raise NotImplementedError("write your pallas kernel here")


# speedup vs baseline: 2.3421x; 2.3421x over previous
import jax
import jax.numpy as jnp
from jax.experimental import pallas as pl


_BLOCK_S = 512


def _add_kernel(x_ref, e_ref, o_ref):
    o_ref[...] = x_ref[...] + e_ref[...]


def kernel(x, encoding):
    N, S, D = x.shape
    x2 = x.reshape(S, D)
    out = pl.pallas_call(
        _add_kernel,
        out_shape=jax.ShapeDtypeStruct((S, D), x.dtype),
        grid=(S // _BLOCK_S,),
        in_specs=[
            pl.BlockSpec((_BLOCK_S, D), lambda i: (i, 0)),
            pl.BlockSpec((_BLOCK_S, D), lambda i: (i, 0)),
        ],
        out_specs=pl.BlockSpec((_BLOCK_S, D), lambda i: (i, 0)),
    )(x2, encoding)
    return out.reshape(N, S, D)


# speedup vs baseline: 2.4095x; 1.0288x over previous
import jax
import jax.numpy as jnp
from jax.experimental import pallas as pl


_BLOCK_S = 1024


def _add_kernel(x_ref, e_ref, o_ref):
    o_ref[...] = x_ref[...] + e_ref[...]


def kernel(x, encoding):
    N, S, D = x.shape
    x2 = x.reshape(S, D)
    out = pl.pallas_call(
        _add_kernel,
        out_shape=jax.ShapeDtypeStruct((S, D), x.dtype),
        grid=(S // _BLOCK_S,),
        in_specs=[
            pl.BlockSpec((_BLOCK_S, D), lambda i: (i, 0)),
            pl.BlockSpec((_BLOCK_S, D), lambda i: (i, 0)),
        ],
        out_specs=pl.BlockSpec((_BLOCK_S, D), lambda i: (i, 0)),
    )(x2, encoding)
    return out.reshape(N, S, D)
